# baseline (speedup 1.0000x reference)
import functools

import jax
import jax.numpy as jnp
from jax import lax
from jax.experimental import pallas as pl
from jax.experimental.pallas import tpu as pltpu
from jax.experimental.pallas import tpu_sc as plsc

N, F, H, E = 10000, 128, 8, 320000
D = F // H
FH = F // 2

NC, NS = 2, 16
NW = NC * NS
NSPLIT = 2
EC = E // NSPLIT

RPT = 624
N_COVERED = RPT * NS
N_REM = N - N_COVERED

_f32 = jnp.float32
_i32 = jnp.int32

_SC_MESH = plsc.VectorSubcoreMesh(core_axis_name="c", subcore_axis_name="s",
                                  num_cores=NC, num_subcores=NS)
_SC_PARAMS = pltpu.CompilerParams(use_tc_tiling_on_sc=False)


def _leaky(x):
    return jnp.where(x >= 0, x, 0.2 * x)


def _bf16_bits(x):
    r = lax.bitcast_convert_type(x, _i32)
    return r + jnp.int32(0x7FFF) + jnp.bitwise_and(jnp.right_shift(r, 16), 1)


def _pack(lo, hi):
    bl = jnp.bitwise_and(jnp.right_shift(_bf16_bits(lo), 16), jnp.int32(0xFFFF))
    bh = jnp.bitwise_and(_bf16_bits(hi), jnp.int32(-65536))
    return jnp.bitwise_or(bh, bl)


def _unpack(p):
    lo = lax.bitcast_convert_type(jnp.left_shift(p, 16), _f32)
    hi = lax.bitcast_convert_type(jnp.bitwise_and(p, jnp.int32(-65536)), _f32)
    return lo, hi


def _node_body(h_ref, wsT, bs, wdT, bd, w1T, b1, w2T, b2, k_ref,
               gsrc_ref, gdst_ref, base_ref):
    h = h_ref[...]
    gs = jnp.dot(h, wsT[...], preferred_element_type=_f32) + bs[...]
    gd = jnp.dot(h, wdT[...], preferred_element_type=_f32) + bd[...]
    gsrc_ref[...] = _pack(gs[:, :FH], gs[:, FH:])
    gdst_ref[...] = _pack(gd[:, :FH], gd[:, FH:])
    x = _leaky(gd)
    u = _leaky(jnp.dot(x, w1T[...], preferred_element_type=_f32) + b1[...])
    t = jnp.dot(u, w2T[...], preferred_element_type=_f32) + b2[...]
    a = jax.nn.sigmoid(t)
    arep = jnp.dot(a, k_ref[...], preferred_element_type=_f32)
    base_ref[...] = arep * gd


def _node_call(h, wsT, bs, wdT, bd, w1T, b1, w2T, b2, k):
    R = 2000
    grid = (N // R,)
    row_spec = pl.BlockSpec((R, F), lambda i: (i, 0))
    half_spec = pl.BlockSpec((R, FH), lambda i: (i, 0))
    full = lambda shape: pl.BlockSpec(shape, lambda i: tuple(0 for _ in shape))
    return pl.pallas_call(
        _node_body,
        grid=grid,
        in_specs=[row_spec, full((F, F)), full((1, F)), full((F, F)), full((1, F)),
                  full((F, 32)), full((1, 32)), full((32, H)), full((1, H)),
                  full((H, F))],
        out_specs=[half_spec, half_spec, row_spec],
        out_shape=[jax.ShapeDtypeStruct((N, FH), _i32),
                   jax.ShapeDtypeStruct((N, FH), _i32),
                   jax.ShapeDtypeStruct((N, F), _f32)],
    )(h, wsT, bs, wdT, bd, w1T, b1, w2T, b2, k)


def _edge_body(ms_ref, md_ref, w1Tlo, w1Thi, b1, w2T, b2, klo_ref, khi_ref,
               w_ref):
    ms_lo, ms_hi = _unpack(ms_ref[...])
    md_lo, md_hi = _unpack(md_ref[...])
    x_lo = _leaky(ms_lo + md_lo)
    x_hi = _leaky(ms_hi + md_hi)
    u = jnp.dot(x_lo, w1Tlo[...], preferred_element_type=_f32)
    u = u + jnp.dot(x_hi, w1Thi[...], preferred_element_type=_f32)
    u = _leaky(u + b1[...])
    t = jnp.dot(u, w2T[...], preferred_element_type=_f32) + b2[...]
    a = jax.nn.sigmoid(t)
    w_lo = jnp.dot(a, klo_ref[...], preferred_element_type=_f32) * ms_lo
    w_hi = jnp.dot(a, khi_ref[...], preferred_element_type=_f32) * ms_hi
    w_ref[...] = jnp.concatenate([w_lo, w_hi], axis=1)


def _edge_call(msrc, mdst, w1Tlo, w1Thi, b1, w2T, b2, klo, khi):
    TE = 2000
    ne = msrc.shape[0]
    half_spec = pl.BlockSpec((TE, FH), lambda i: (i, 0))
    out_spec = pl.BlockSpec((TE, F), lambda i: (i, 0))
    full = lambda shape: pl.BlockSpec(shape, lambda i: tuple(0 for _ in shape))
    return pl.pallas_call(
        _edge_body,
        grid=(ne // TE,),
        in_specs=[half_spec, half_spec, full((FH, 32)), full((FH, 32)),
                  full((1, 32)), full((32, H)), full((1, H)),
                  full((H, FH)), full((H, FH))],
        out_specs=out_spec,
        out_shape=jax.ShapeDtypeStruct((ne, F), _f32),
    )(msrc, mdst, w1Tlo, w1Thi, b1, w2T, b2, klo, khi)


def _combine_body(b_ref, p00, p01, p10, p11, o_ref):
    o_ref[...] = (b_ref[...] + p00[...] + p01[...]) + (p10[...] + p11[...])


def _combine_call(base, partials):
    R = 2000
    row_spec = pl.BlockSpec((R, F), lambda i: (i, 0))
    return pl.pallas_call(
        _combine_body,
        grid=(N // R,),
        in_specs=[row_spec] * 5,
        out_specs=row_spec,
        out_shape=jax.ShapeDtypeStruct((N, F), _f32),
    )(base, *partials)


def _make_gather(ec, ch, nb):
    epw = ec // NW
    n_chunk = epw // ch
    n_group = n_chunk // nb
    assert epw % ch == 0 and n_chunk % nb == 0 and ch % 8 == 0

    @functools.partial(
        pl.kernel,
        out_type=[jax.ShapeDtypeStruct((ec, FH), _i32),
                  jax.ShapeDtypeStruct((ec, FH), _i32)],
        mesh=_SC_MESH,
        scratch_types=[
            pltpu.VMEM((epw,), _i32),
            pltpu.VMEM((epw,), _i32),
            pltpu.VMEM((nb, ch, FH), _i32),
            pltpu.VMEM((nb, ch, FH), _i32),
        ] + [pltpu.SemaphoreType.DMA] * (4 * nb),
        compiler_params=_SC_PARAMS,
    )
    def gather_kernel(src_hbm, dst_hbm, gsrc_hbm, gdst_hbm, msrc_hbm, mdst_hbm,
                      idxs_all, idxd_all, rows_s, rows_d, *sems):
        gs = sems[0:nb]
        gd = sems[nb:2 * nb]
        ss = sems[2 * nb:3 * nb]
        sd = sems[3 * nb:4 * nb]
        wid = lax.axis_index("s") * NC + lax.axis_index("c")
        ebase = wid * epw

        pltpu.sync_copy(src_hbm.at[pl.ds(ebase, epw)], idxs_all)
        pltpu.sync_copy(dst_hbm.at[pl.ds(ebase, epw)], idxd_all)

        def fire_gather(k, b):
            off = k * ch
            pltpu.async_copy(gsrc_hbm.at[idxs_all.at[pl.ds(off, ch)]],
                             rows_s.at[b], gs[b])
            pltpu.async_copy(gdst_hbm.at[idxd_all.at[pl.ds(off, ch)]],
                             rows_d.at[b], gd[b])

        def wait_gather(k, b):
            off = k * ch
            pltpu.make_async_copy(gsrc_hbm.at[idxs_all.at[pl.ds(off, ch)]],
                                  rows_s.at[b], gs[b]).wait()
            pltpu.make_async_copy(gdst_hbm.at[idxd_all.at[pl.ds(off, ch)]],
                                  rows_d.at[b], gd[b]).wait()

        def fire_store(k, b):
            off = ebase + k * ch
            pltpu.async_copy(rows_s.at[b], msrc_hbm.at[pl.ds(off, ch)], ss[b])
            pltpu.async_copy(rows_d.at[b], mdst_hbm.at[pl.ds(off, ch)], sd[b])

        def wait_store(k, b):
            off = ebase + k * ch
            pltpu.make_async_copy(rows_s.at[b], msrc_hbm.at[pl.ds(off, ch)],
                                  ss[b]).wait()
            pltpu.make_async_copy(rows_d.at[b], mdst_hbm.at[pl.ds(off, ch)],
                                  sd[b]).wait()

        fire_gather(0, 0)
        fire_gather(1, 1)

        def group(g, carry):
            for b in range(nb):
                k = g * nb + b
                slot_next = (b + 2) % nb

                @pl.when(k >= nb - 2)
                def _():
                    wait_store(k, slot_next)

                @pl.when(k + 2 < n_chunk)
                def _():
                    fire_gather(k + 2, slot_next)

                wait_gather(k, b)
                fire_store(k, b)
            return carry

        lax.fori_loop(0, n_group, group, 0)

        for k in range(n_chunk - (nb - 2), n_chunk):
            wait_store(k, k % nb)

    return gather_kernel


def _make_scatter(ec, ch, nb):
    epw = ec // NW
    n_chunk = epw // ch
    n_group = n_chunk // nb
    assert epw % ch == 0 and n_chunk % nb == 0 and ch % 8 == 0

    @functools.partial(
        pl.kernel,
        out_type=jax.ShapeDtypeStruct((NC, N, F), _f32),
        mesh=_SC_MESH,
        scratch_types=[
            pltpu.VMEM_SHARED((N, F), _f32),
        ] + [pltpu.VMEM((ch,), _i32)] * nb + [
            pltpu.VMEM((nb, ch, F), _f32),
        ] + [pltpu.SemaphoreType.DMA] * (3 * nb),
    )
    def scatter_kernel(dst_hbm, w_hbm, zeros_hbm, out_hbm, acc, *rest):
        idxs = rest[0:nb]
        rows = rest[nb]
        sems = rest[nb + 1:]
        li = sems[0:nb]
        lr = sems[nb:2 * nb]
        sc = sems[2 * nb:3 * nb]
        c = lax.axis_index("c")
        s = lax.axis_index("s")
        wid = s * NC + c
        ebase = wid * epw
        rbase = s * RPT

        pltpu.sync_copy(zeros_hbm.at[pl.ds(rbase, RPT)],
                        acc.at[pl.ds(rbase, RPT)])

        @pl.when(s == 0)
        def _():
            pltpu.sync_copy(zeros_hbm.at[pl.ds(N_COVERED, N_REM)],
                            acc.at[pl.ds(N_COVERED, N_REM)])

        plsc.subcore_barrier()

        def fire_load(k, b):
            off = ebase + k * ch
            pltpu.async_copy(dst_hbm.at[pl.ds(off, ch)], idxs[b], li[b])
            pltpu.async_copy(w_hbm.at[pl.ds(off, ch)], rows.at[b], lr[b])

        def wait_load(k, b):
            off = ebase + k * ch
            pltpu.make_async_copy(dst_hbm.at[pl.ds(off, ch)], idxs[b],
                                  li[b]).wait()
            pltpu.make_async_copy(w_hbm.at[pl.ds(off, ch)], rows.at[b],
                                  lr[b]).wait()

        def fire_scatter(b):
            pltpu.async_copy(rows.at[b], acc.at[idxs[b]], sc[b], add=True)

        def wait_scatter(b):
            pltpu.make_async_copy(rows.at[b], acc.at[idxs[b]], sc[b]).wait()

        fire_load(0, 0)
        fire_load(1, 1)

        def group(g, carry):
            for b in range(nb):
                k = g * nb + b
                slot_next = (b + 2) % nb

                @pl.when(k >= nb - 2)
                def _():
                    wait_scatter(slot_next)

                @pl.when(k + 2 < n_chunk)
                def _():
                    fire_load(k + 2, slot_next)

                wait_load(k, b)
                fire_scatter(b)
            return carry

        lax.fori_loop(0, n_group, group, 0)

        for k in range(n_chunk - (nb - 2), n_chunk):
            wait_scatter(k % nb)

        plsc.subcore_barrier()
        pltpu.sync_copy(acc.at[pl.ds(rbase, RPT)],
                        out_hbm.at[c, pl.ds(rbase, RPT)])

        @pl.when(s == 0)
        def _():
            pltpu.sync_copy(acc.at[pl.ds(N_COVERED, N_REM)],
                            out_hbm.at[c, pl.ds(N_COVERED, N_REM)])

    return scatter_kernel


_gather_half = _make_gather(EC, ch=40, nb=5)
_scatter_half = _make_scatter(EC, ch=40, nb=5)


def kernel(h, edge_index, W_src_w, W_src_b, W_dst_w, W_dst_b,
           asrc_w1, asrc_b1, asrc_w2, asrc_b2,
           adst_w1, adst_b1, adst_w2, adst_b2):
    src = edge_index[0]
    dst = edge_index[1]
    k = jnp.kron(jnp.eye(H, dtype=_f32), jnp.ones((1, D), _f32))
    klo, khi = k[:, :FH], k[:, FH:]
    zeros = jnp.zeros((N, F), _f32)

    gsrc, gdst, base = _node_call(
        h, W_src_w.T, W_src_b.reshape(1, F), W_dst_w.T, W_dst_b.reshape(1, F),
        adst_w1.T, adst_b1.reshape(1, 32), adst_w2.T, adst_b2.reshape(1, H), k)

    w1T = asrc_w1.T
    b1 = asrc_b1.reshape(1, 32)
    w2T = asrc_w2.T
    b2 = asrc_b2.reshape(1, H)

    partials = []
    msgs = [_gather_half(src[i * EC:(i + 1) * EC], dst[i * EC:(i + 1) * EC],
                         gsrc, gdst) for i in range(NSPLIT)]
    for i in range(NSPLIT):
        msrc, mdst = msgs[i]
        weighted = _edge_call(msrc, mdst, w1T[:FH], w1T[FH:], b1, w2T, b2,
                              klo, khi)
        p = _scatter_half(dst[i * EC:(i + 1) * EC], weighted, zeros)
        partials.extend([p[0], p[1]])

    return _combine_call(base, partials)


# speedup vs baseline: 1.3066x; 1.3066x over previous
import functools

import jax
import jax.numpy as jnp
from jax import lax
from jax.experimental import pallas as pl
from jax.experimental.pallas import tpu as pltpu
from jax.experimental.pallas import tpu_sc as plsc

N, F, H, E = 10000, 128, 8, 320000
D = F // H

NC, NS = 2, 16
NW = NC * NS
EC_SPLITS = (128000, 192000)
EC_CFG = {128000: dict(ch=80, nb=5), 192000: dict(ch=48, nb=5)}
EC_SCFG = {128000: dict(ch=16, nb=5), 192000: dict(ch=48, nb=5)}

RPT = 624
N_COVERED = RPT * NS
N_REM = N - N_COVERED

_f32 = jnp.float32
_i32 = jnp.int32

_SC_MESH = plsc.VectorSubcoreMesh(core_axis_name="c", subcore_axis_name="s",
                                  num_cores=NC, num_subcores=NS)


def _leaky(x):
    return jnp.where(x >= 0, x, 0.2 * x)


def _node_body(h_ref, wsT, bs, wdT, bd, w1T, b1, w2T, b2, k_ref,
               gsrc_ref, gdst_ref, base_ref):
    h = h_ref[...]
    gs = jnp.dot(h, wsT[...], preferred_element_type=_f32) + bs[...]
    gd = jnp.dot(h, wdT[...], preferred_element_type=_f32) + bd[...]
    gsrc_ref[...] = gs
    gdst_ref[...] = gd
    x = _leaky(gd)
    u = _leaky(jnp.dot(x, w1T[...], preferred_element_type=_f32) + b1[...])
    t = jnp.dot(u, w2T[...], preferred_element_type=_f32) + b2[...]
    a = jax.nn.sigmoid(t)
    arep = jnp.dot(a, k_ref[...], preferred_element_type=_f32)
    base_ref[...] = arep * gd


def _node_call(h, wsT, bs, wdT, bd, w1T, b1, w2T, b2, k):
    R = 2000
    grid = (N // R,)
    row_spec = pl.BlockSpec((R, F), lambda i: (i, 0))
    full = lambda shape: pl.BlockSpec(shape, lambda i: tuple(0 for _ in shape))
    return pl.pallas_call(
        _node_body,
        grid=grid,
        in_specs=[row_spec, full((F, F)), full((1, F)), full((F, F)), full((1, F)),
                  full((F, 32)), full((1, 32)), full((32, H)), full((1, H)),
                  full((H, F))],
        out_specs=[row_spec, row_spec, row_spec],
        out_shape=[jax.ShapeDtypeStruct((N, F), _f32)] * 3,
    )(h, wsT, bs, wdT, bd, w1T, b1, w2T, b2, k)


def _edge_body(ms_ref, md_ref, w1T, b1, w2T, b2, k_ref, w_ref):
    ms = ms_ref[...]
    s = ms + md_ref[...]
    x = _leaky(s)
    u = _leaky(jnp.dot(x, w1T[...], preferred_element_type=_f32) + b1[...])
    t = jnp.dot(u, w2T[...], preferred_element_type=_f32) + b2[...]
    a = jax.nn.sigmoid(t)
    arep = jnp.dot(a, k_ref[...], preferred_element_type=_f32)
    w_ref[...] = arep * ms


def _edge_call(msrc, mdst, w1T, b1, w2T, b2, k):
    TE = 2000
    ne = msrc.shape[0]
    grid = (ne // TE,)
    row_spec = pl.BlockSpec((TE, F), lambda i: (i, 0))
    full = lambda shape: pl.BlockSpec(shape, lambda i: tuple(0 for _ in shape))
    return pl.pallas_call(
        _edge_body,
        grid=grid,
        in_specs=[row_spec, row_spec, full((F, 32)), full((1, 32)),
                  full((32, H)), full((1, H)), full((H, F))],
        out_specs=row_spec,
        out_shape=jax.ShapeDtypeStruct((ne, F), _f32),
    )(msrc, mdst, w1T, b1, w2T, b2, k)


def _combine_body(b_ref, p00, p01, p10, p11, o_ref):
    o_ref[...] = (b_ref[...] + p00[...] + p01[...]) + (p10[...] + p11[...])


def _combine_call(base, partials):
    R = 2000
    row_spec = pl.BlockSpec((R, F), lambda i: (i, 0))
    return pl.pallas_call(
        _combine_body,
        grid=(N // R,),
        in_specs=[row_spec] * 5,
        out_specs=row_spec,
        out_shape=jax.ShapeDtypeStruct((N, F), _f32),
    )(base, *partials)


def _make_gather(ec, ch, nb):
    epw = ec // NW
    n_chunk = epw // ch
    n_group = n_chunk // nb
    assert epw % ch == 0 and n_chunk % nb == 0 and ch % 8 == 0

    @functools.partial(
        pl.kernel,
        out_type=[jax.ShapeDtypeStruct((ec, F), _f32),
                  jax.ShapeDtypeStruct((ec, F), _f32)],
        mesh=_SC_MESH,
        scratch_types=[
            pltpu.VMEM((epw,), _i32),
            pltpu.VMEM((epw,), _i32),
            pltpu.VMEM((nb, ch, F), _f32),
            pltpu.VMEM((nb, ch, F), _f32),
        ] + [pltpu.SemaphoreType.DMA] * (4 * nb),
    )
    def gather_kernel(src_hbm, dst_hbm, gsrc_hbm, gdst_hbm, msrc_hbm, mdst_hbm,
                      idxs_all, idxd_all, rows_s, rows_d, *sems):
        gs = sems[0:nb]
        gd = sems[nb:2 * nb]
        ss = sems[2 * nb:3 * nb]
        sd = sems[3 * nb:4 * nb]
        wid = lax.axis_index("s") * NC + lax.axis_index("c")
        ebase = wid * epw

        pltpu.sync_copy(src_hbm.at[pl.ds(ebase, epw)], idxs_all)
        pltpu.sync_copy(dst_hbm.at[pl.ds(ebase, epw)], idxd_all)

        def fire_gather(k, b):
            off = k * ch
            pltpu.async_copy(gsrc_hbm.at[idxs_all.at[pl.ds(off, ch)]],
                             rows_s.at[b], gs[b])
            pltpu.async_copy(gdst_hbm.at[idxd_all.at[pl.ds(off, ch)]],
                             rows_d.at[b], gd[b])

        def wait_gather(k, b):
            off = k * ch
            pltpu.make_async_copy(gsrc_hbm.at[idxs_all.at[pl.ds(off, ch)]],
                                  rows_s.at[b], gs[b]).wait()
            pltpu.make_async_copy(gdst_hbm.at[idxd_all.at[pl.ds(off, ch)]],
                                  rows_d.at[b], gd[b]).wait()

        def fire_store(k, b):
            off = ebase + k * ch
            pltpu.async_copy(rows_s.at[b], msrc_hbm.at[pl.ds(off, ch)], ss[b])
            pltpu.async_copy(rows_d.at[b], mdst_hbm.at[pl.ds(off, ch)], sd[b])

        def wait_store(k, b):
            off = ebase + k * ch
            pltpu.make_async_copy(rows_s.at[b], msrc_hbm.at[pl.ds(off, ch)],
                                  ss[b]).wait()
            pltpu.make_async_copy(rows_d.at[b], mdst_hbm.at[pl.ds(off, ch)],
                                  sd[b]).wait()

        fire_gather(0, 0)
        fire_gather(1, 1)

        def group(g, carry):
            for b in range(nb):
                k = g * nb + b
                slot_next = (b + 2) % nb

                @pl.when(k >= nb - 2)
                def _():
                    wait_store(k, slot_next)

                @pl.when(k + 2 < n_chunk)
                def _():
                    fire_gather(k + 2, slot_next)

                wait_gather(k, b)
                fire_store(k, b)
            return carry

        lax.fori_loop(0, n_group, group, 0)

        for k in range(n_chunk - (nb - 2), n_chunk):
            wait_store(k, k % nb)

    return gather_kernel


def _make_scatter(ec, ch, nb):
    epw = ec // NW
    n_chunk = epw // ch
    n_group = n_chunk // nb
    assert epw % ch == 0 and n_chunk % nb == 0 and ch % 8 == 0

    @functools.partial(
        pl.kernel,
        out_type=jax.ShapeDtypeStruct((NC, N, F), _f32),
        mesh=_SC_MESH,
        scratch_types=[
            pltpu.VMEM_SHARED((N, F), _f32),
        ] + [pltpu.VMEM((ch,), _i32)] * nb + [
            pltpu.VMEM((nb, ch, F), _f32),
        ] + [pltpu.SemaphoreType.DMA] * (3 * nb),
    )
    def scatter_kernel(dst_hbm, w_hbm, zeros_hbm, out_hbm, acc, *rest):
        idxs = rest[0:nb]
        rows = rest[nb]
        sems = rest[nb + 1:]
        li = sems[0:nb]
        lr = sems[nb:2 * nb]
        sc = sems[2 * nb:3 * nb]
        c = lax.axis_index("c")
        s = lax.axis_index("s")
        wid = s * NC + c
        ebase = wid * epw
        rbase = s * RPT

        pltpu.sync_copy(zeros_hbm.at[pl.ds(rbase, RPT)],
                        acc.at[pl.ds(rbase, RPT)])

        @pl.when(s == 0)
        def _():
            pltpu.sync_copy(zeros_hbm.at[pl.ds(N_COVERED, N_REM)],
                            acc.at[pl.ds(N_COVERED, N_REM)])

        plsc.subcore_barrier()

        def fire_load(k, b):
            off = ebase + k * ch
            pltpu.async_copy(dst_hbm.at[pl.ds(off, ch)], idxs[b], li[b])
            pltpu.async_copy(w_hbm.at[pl.ds(off, ch)], rows.at[b], lr[b])

        def wait_load(k, b):
            off = ebase + k * ch
            pltpu.make_async_copy(dst_hbm.at[pl.ds(off, ch)], idxs[b],
                                  li[b]).wait()
            pltpu.make_async_copy(w_hbm.at[pl.ds(off, ch)], rows.at[b],
                                  lr[b]).wait()

        def fire_scatter(b):
            pltpu.async_copy(rows.at[b], acc.at[idxs[b]], sc[b], add=True)

        def wait_scatter(b):
            pltpu.make_async_copy(rows.at[b], acc.at[idxs[b]], sc[b]).wait()

        fire_load(0, 0)
        fire_load(1, 1)

        def group(g, carry):
            for b in range(nb):
                k = g * nb + b
                slot_next = (b + 2) % nb

                @pl.when(k >= nb - 2)
                def _():
                    wait_scatter(slot_next)

                @pl.when(k + 2 < n_chunk)
                def _():
                    fire_load(k + 2, slot_next)

                wait_load(k, b)
                fire_scatter(b)
            return carry

        lax.fori_loop(0, n_group, group, 0)

        for k in range(n_chunk - (nb - 2), n_chunk):
            wait_scatter(k % nb)

        plsc.subcore_barrier()
        pltpu.sync_copy(acc.at[pl.ds(rbase, RPT)],
                        out_hbm.at[c, pl.ds(rbase, RPT)])

        @pl.when(s == 0)
        def _():
            pltpu.sync_copy(acc.at[pl.ds(N_COVERED, N_REM)],
                            out_hbm.at[c, pl.ds(N_COVERED, N_REM)])

    return scatter_kernel


_gathers = tuple(_make_gather(ec, **EC_CFG[ec]) for ec in EC_SPLITS)
_scatters = tuple(_make_scatter(ec, **EC_SCFG[ec]) for ec in EC_SPLITS)


def kernel(h, edge_index, W_src_w, W_src_b, W_dst_w, W_dst_b,
           asrc_w1, asrc_b1, asrc_w2, asrc_b2,
           adst_w1, adst_b1, adst_w2, adst_b2):
    src = edge_index[0]
    dst = edge_index[1]
    k = jnp.kron(jnp.eye(H, dtype=_f32), jnp.ones((1, D), _f32))
    zeros = jnp.zeros((N, F), _f32)

    gsrc, gdst, base = _node_call(
        h, W_src_w.T, W_src_b.reshape(1, F), W_dst_w.T, W_dst_b.reshape(1, F),
        adst_w1.T, adst_b1.reshape(1, 32), adst_w2.T, adst_b2.reshape(1, H), k)

    w1T = asrc_w1.T
    b1 = asrc_b1.reshape(1, 32)
    w2T = asrc_w2.T
    b2 = asrc_b2.reshape(1, H)

    bounds = (0, EC_SPLITS[0], E)
    partials = []
    msgs = [_gathers[i](src[bounds[i]:bounds[i + 1]],
                        dst[bounds[i]:bounds[i + 1]], gsrc, gdst)
            for i in range(2)]
    for i in range(2):
        msrc, mdst = msgs[i]
        weighted = _edge_call(msrc, mdst, w1T, b1, w2T, b2, k)
        p = _scatters[i](dst[bounds[i]:bounds[i + 1]], weighted, zeros)
        partials.extend([p[0], p[1]])

    return _combine_call(base, partials)


# speedup vs baseline: 1.3618x; 1.0422x over previous
import functools

import jax
import jax.numpy as jnp
from jax import lax
from jax.experimental import pallas as pl
from jax.experimental.pallas import tpu as pltpu
from jax.experimental.pallas import tpu_sc as plsc

N, F, H, E = 10000, 128, 8, 320000
D = F // H

NC, NS = 2, 16
NW = NC * NS
NSPLIT = 2
EC = E // NSPLIT

RPT = 624
N_COVERED = RPT * NS
N_REM = N - N_COVERED

_f32 = jnp.float32
_i32 = jnp.int32

_SC_MESH = plsc.VectorSubcoreMesh(core_axis_name="c", subcore_axis_name="s",
                                  num_cores=NC, num_subcores=NS)


def _leaky(x):
    return jnp.where(x >= 0, x, 0.2 * x)


def _node_body(h_ref, wsT, bs, wdT, bd, w1T, b1, w2T, b2, k_ref,
               gsrc_ref, gdst_ref, base_ref):
    h = h_ref[...]
    gs = jnp.dot(h, wsT[...], preferred_element_type=_f32) + bs[...]
    gd = jnp.dot(h, wdT[...], preferred_element_type=_f32) + bd[...]
    gsrc_ref[...] = gs
    gdst_ref[...] = gd
    x = _leaky(gd)
    u = _leaky(jnp.dot(x, w1T[...], preferred_element_type=_f32) + b1[...])
    t = jnp.dot(u, w2T[...], preferred_element_type=_f32) + b2[...]
    a = jax.nn.sigmoid(t)
    arep = jnp.dot(a, k_ref[...], preferred_element_type=_f32)
    base_ref[...] = arep * gd


def _node_call(h, wsT, bs, wdT, bd, w1T, b1, w2T, b2, k):
    R = 2000
    grid = (N // R,)
    row_spec = pl.BlockSpec((R, F), lambda i: (i, 0))
    full = lambda shape: pl.BlockSpec(shape, lambda i: tuple(0 for _ in shape))
    return pl.pallas_call(
        _node_body,
        grid=grid,
        in_specs=[row_spec, full((F, F)), full((1, F)), full((F, F)), full((1, F)),
                  full((F, 32)), full((1, 32)), full((32, H)), full((1, H)),
                  full((H, F))],
        out_specs=[row_spec, row_spec, row_spec],
        out_shape=[jax.ShapeDtypeStruct((N, F), _f32)] * 3,
    )(h, wsT, bs, wdT, bd, w1T, b1, w2T, b2, k)


def _edge_body(ms_ref, md_ref, w1T, b1, w2T, b2, k_ref, w_ref):
    ms = ms_ref[...]
    s = ms + md_ref[...]
    x = _leaky(s)
    u = _leaky(jnp.dot(x, w1T[...], preferred_element_type=_f32) + b1[...])
    t = jnp.dot(u, w2T[...], preferred_element_type=_f32) + b2[...]
    a = jax.nn.sigmoid(t)
    arep = jnp.dot(a, k_ref[...], preferred_element_type=_f32)
    w_ref[...] = arep * ms


def _edge_call(msrc, mdst, w1T, b1, w2T, b2, k):
    TE = 2000
    ne = msrc.shape[0]
    grid = (ne // TE,)
    row_spec = pl.BlockSpec((TE, F), lambda i: (i, 0))
    full = lambda shape: pl.BlockSpec(shape, lambda i: tuple(0 for _ in shape))
    return pl.pallas_call(
        _edge_body,
        grid=grid,
        in_specs=[row_spec, row_spec, full((F, 32)), full((1, 32)),
                  full((32, H)), full((1, H)), full((H, F))],
        out_specs=row_spec,
        out_shape=jax.ShapeDtypeStruct((ne, F), _f32),
    )(msrc, mdst, w1T, b1, w2T, b2, k)


def _combine_body(b_ref, p00, p01, p10, p11, o_ref):
    o_ref[...] = (b_ref[...] + p00[...] + p01[...]) + (p10[...] + p11[...])


def _combine_call(base, partials):
    R = 2000
    row_spec = pl.BlockSpec((R, F), lambda i: (i, 0))
    return pl.pallas_call(
        _combine_body,
        grid=(N // R,),
        in_specs=[row_spec] * 5,
        out_specs=row_spec,
        out_shape=jax.ShapeDtypeStruct((N, F), _f32),
    )(base, *partials)


def _make_gather(ec, ch, nb):
    epw = ec // NW
    n_chunk = epw // ch
    n_group = n_chunk // nb
    assert epw % ch == 0 and n_chunk % nb == 0 and ch % 8 == 0

    @functools.partial(
        pl.kernel,
        out_type=[jax.ShapeDtypeStruct((ec, F), _f32),
                  jax.ShapeDtypeStruct((ec, F), _f32)],
        mesh=_SC_MESH,
        scratch_types=[
            pltpu.VMEM((epw,), _i32),
            pltpu.VMEM((epw,), _i32),
            pltpu.VMEM((nb, ch, F), _f32),
            pltpu.VMEM((nb, ch, F), _f32),
        ] + [pltpu.SemaphoreType.DMA] * (4 * nb),
    )
    def gather_kernel(src_hbm, dst_hbm, gsrc_hbm, gdst_hbm, msrc_hbm, mdst_hbm,
                      idxs_all, idxd_all, rows_s, rows_d, *sems):
        gs = sems[0:nb]
        gd = sems[nb:2 * nb]
        ss = sems[2 * nb:3 * nb]
        sd = sems[3 * nb:4 * nb]
        wid = lax.axis_index("s") * NC + lax.axis_index("c")
        ebase = wid * epw

        pltpu.sync_copy(src_hbm.at[pl.ds(ebase, epw)], idxs_all)
        pltpu.sync_copy(dst_hbm.at[pl.ds(ebase, epw)], idxd_all)

        def fire_gather(k, b):
            off = k * ch
            pltpu.async_copy(gsrc_hbm.at[idxs_all.at[pl.ds(off, ch)]],
                             rows_s.at[b], gs[b])
            pltpu.async_copy(gdst_hbm.at[idxd_all.at[pl.ds(off, ch)]],
                             rows_d.at[b], gd[b])

        def wait_gather(k, b):
            off = k * ch
            pltpu.make_async_copy(gsrc_hbm.at[idxs_all.at[pl.ds(off, ch)]],
                                  rows_s.at[b], gs[b]).wait()
            pltpu.make_async_copy(gdst_hbm.at[idxd_all.at[pl.ds(off, ch)]],
                                  rows_d.at[b], gd[b]).wait()

        def fire_store(k, b):
            off = ebase + k * ch
            pltpu.async_copy(rows_s.at[b], msrc_hbm.at[pl.ds(off, ch)], ss[b])
            pltpu.async_copy(rows_d.at[b], mdst_hbm.at[pl.ds(off, ch)], sd[b])

        def wait_store(k, b):
            off = ebase + k * ch
            pltpu.make_async_copy(rows_s.at[b], msrc_hbm.at[pl.ds(off, ch)],
                                  ss[b]).wait()
            pltpu.make_async_copy(rows_d.at[b], mdst_hbm.at[pl.ds(off, ch)],
                                  sd[b]).wait()

        fire_gather(0, 0)
        fire_gather(1, 1)

        def group(g, carry):
            for b in range(nb):
                k = g * nb + b
                slot_next = (b + 2) % nb

                @pl.when(k >= nb - 2)
                def _():
                    wait_store(k, slot_next)

                @pl.when(k + 2 < n_chunk)
                def _():
                    fire_gather(k + 2, slot_next)

                wait_gather(k, b)
                fire_store(k, b)
            return carry

        lax.fori_loop(0, n_group, group, 0)

        for k in range(n_chunk - (nb - 2), n_chunk):
            wait_store(k, k % nb)

    return gather_kernel


def _make_scatter(ec, ch, nb):
    epw = ec // NW
    n_chunk = epw // ch
    n_group = n_chunk // nb
    assert epw % ch == 0 and n_chunk % nb == 0 and ch % 8 == 0

    @functools.partial(
        pl.kernel,
        out_type=jax.ShapeDtypeStruct((NC, N, F), _f32),
        mesh=_SC_MESH,
        scratch_types=[
            pltpu.VMEM_SHARED((N, F), _f32),
        ] + [pltpu.VMEM((ch,), _i32)] * nb + [
            pltpu.VMEM((nb, ch, F), _f32),
        ] + [pltpu.SemaphoreType.DMA] * (3 * nb),
    )
    def scatter_kernel(dst_hbm, w_hbm, zeros_hbm, out_hbm, acc, *rest):
        idxs = rest[0:nb]
        rows = rest[nb]
        sems = rest[nb + 1:]
        li = sems[0:nb]
        lr = sems[nb:2 * nb]
        sc = sems[2 * nb:3 * nb]
        c = lax.axis_index("c")
        s = lax.axis_index("s")
        wid = s * NC + c
        ebase = wid * epw
        rbase = s * RPT

        pltpu.sync_copy(zeros_hbm.at[pl.ds(rbase, RPT)],
                        acc.at[pl.ds(rbase, RPT)])

        @pl.when(s == 0)
        def _():
            pltpu.sync_copy(zeros_hbm.at[pl.ds(N_COVERED, N_REM)],
                            acc.at[pl.ds(N_COVERED, N_REM)])

        plsc.subcore_barrier()

        def fire_load(k, b):
            off = ebase + k * ch
            pltpu.async_copy(dst_hbm.at[pl.ds(off, ch)], idxs[b], li[b])
            pltpu.async_copy(w_hbm.at[pl.ds(off, ch)], rows.at[b], lr[b])

        def wait_load(k, b):
            off = ebase + k * ch
            pltpu.make_async_copy(dst_hbm.at[pl.ds(off, ch)], idxs[b],
                                  li[b]).wait()
            pltpu.make_async_copy(w_hbm.at[pl.ds(off, ch)], rows.at[b],
                                  lr[b]).wait()

        def fire_scatter(b):
            pltpu.async_copy(rows.at[b], acc.at[idxs[b]], sc[b], add=True)

        def wait_scatter(b):
            pltpu.make_async_copy(rows.at[b], acc.at[idxs[b]], sc[b]).wait()

        fire_load(0, 0)
        fire_load(1, 1)

        def group(g, carry):
            for b in range(nb):
                k = g * nb + b
                slot_next = (b + 2) % nb

                @pl.when(k >= nb - 2)
                def _():
                    wait_scatter(slot_next)

                @pl.when(k + 2 < n_chunk)
                def _():
                    fire_load(k + 2, slot_next)

                wait_load(k, b)
                fire_scatter(b)
            return carry

        lax.fori_loop(0, n_group, group, 0)

        for k in range(n_chunk - (nb - 2), n_chunk):
            wait_scatter(k % nb)

        plsc.subcore_barrier()
        pltpu.sync_copy(acc.at[pl.ds(rbase, RPT)],
                        out_hbm.at[c, pl.ds(rbase, RPT)])

        @pl.when(s == 0)
        def _():
            pltpu.sync_copy(acc.at[pl.ds(N_COVERED, N_REM)],
                            out_hbm.at[c, pl.ds(N_COVERED, N_REM)])

    return scatter_kernel


_gather_half = _make_gather(EC, ch=40, nb=5)
_scatter_half = _make_scatter(EC, ch=40, nb=5)


def kernel(h, edge_index, W_src_w, W_src_b, W_dst_w, W_dst_b,
           asrc_w1, asrc_b1, asrc_w2, asrc_b2,
           adst_w1, adst_b1, adst_w2, adst_b2):
    src = edge_index[0]
    dst = edge_index[1]
    k = jnp.kron(jnp.eye(H, dtype=_f32), jnp.ones((1, D), _f32))
    zeros = jnp.zeros((N, F), _f32)

    gsrc, gdst, base = _node_call(
        h, W_src_w.T, W_src_b.reshape(1, F), W_dst_w.T, W_dst_b.reshape(1, F),
        adst_w1.T, adst_b1.reshape(1, 32), adst_w2.T, adst_b2.reshape(1, H), k)

    w1T = asrc_w1.T
    b1 = asrc_b1.reshape(1, 32)
    w2T = asrc_w2.T
    b2 = asrc_b2.reshape(1, H)

    partials = []
    msgs = [_gather_half(src[i * EC:(i + 1) * EC], dst[i * EC:(i + 1) * EC],
                         gsrc, gdst) for i in range(NSPLIT)]
    for i in range(NSPLIT):
        msrc, mdst = msgs[i]
        weighted = _edge_call(msrc, mdst, w1T, b1, w2T, b2, k)
        p = _scatter_half(dst[i * EC:(i + 1) * EC], weighted, zeros)
        partials.extend([p[0], p[1]])

    return _combine_call(base, partials)
